# baseline (device time: 19011 ns/iter reference)
import os

import jax
import jax.numpy as jnp
from jax import lax
from jax.experimental import pallas as pl
from jax.experimental.pallas import tpu as pltpu

_KMODE = os.environ.get("KMODE", "full")

N_DEV = 8
ROWS = 512
D_MODEL = 256
D_FF = 512
N_EXP = 16
EXP_PER_DEV = N_EXP // N_DEV
N_CSUB = 4
CSUB = D_FF // N_CSUB

STREAMS = ((0, 160), (160, 176), (336, 176))
N_STREAMS = 3
RBUF_BASE = []
_off = 0
for _b, _r in STREAMS:
    RBUF_BASE.append(tuple(_off + t * _r for t in range(3)))
    _off += 3 * _r
RBUF_ROWS = _off

SUBS = tuple((s, c) for c in range(N_CSUB) for s in range(N_STREAMS))


def kernel(x, router_W, route_idx, expert_W):
    def body(
        x_ref,
        rw_ref,
        idx_ref,
        ew_ref,
        out_ref,
        pbuf,
        rbuf,
        send_sems,
        recv_sems,
    ):
        my = lax.axis_index("i")
        partners = (my ^ 1, my ^ 3, my ^ 4)
        orders = ((0, 1, 2), (1, 2, 0), (2, 0, 1))

        if _KMODE == "min":
            out_ref[:, :] = jnp.zeros((ROWS, D_FF), jnp.float32)
            return

        if _KMODE != "nobarrier":
            barrier_sem = pltpu.get_barrier_semaphore()
            if _KMODE == "withbarrier":
                for prt in partners:
                    pltpu.semaphore_signal(
                        barrier_sem,
                        inc=1,
                        device_id=(prt,),
                        device_id_type=pltpu.DeviceIdType.MESH,
                    )
            else:
                pltpu.semaphore_signal(barrier_sem, inc=1)
                pltpu.semaphore_wait(barrier_sem, 1)

        x_v = x_ref[:, :]
        scores = jnp.dot(x_v, rw_ref[:, :], preferred_element_type=jnp.float32)
        s_max = jnp.max(scores, axis=-1, keepdims=True)
        p = jnp.exp(scores - s_max)
        p = p / jnp.sum(p, axis=-1, keepdims=True)

        idx0 = idx_ref[:, 0:1]
        idx1 = idx_ref[:, 1:2]
        e_iota = lax.broadcasted_iota(jnp.int32, (ROWS, N_EXP), 1)
        g0 = jnp.sum(jnp.where(idx0 == e_iota, p, 0.0), axis=-1, keepdims=True)
        g1 = jnp.sum(jnp.where(idx1 == e_iota, p, 0.0), axis=-1, keepdims=True)
        gs = g0 + g1
        w0 = g0 / gs
        w1 = g1 / gs

        e_base = my * EXP_PER_DEV
        ew_bf = [
            ew_ref[k, :, :].astype(jnp.bfloat16) for k in range(EXP_PER_DEV)
        ]

        def compute_stream(s):
            ro, nrows = STREAMS[s]
            xs = x_v[ro : ro + nrows, :]
            i0 = idx0[ro : ro + nrows, :]
            i1 = idx1[ro : ro + nrows, :]
            v0 = w0[ro : ro + nrows, :]
            v1 = w1[ro : ro + nrows, :]
            acc = None
            for k in range(EXP_PER_DEV):
                e = e_base + k
                m = jnp.where(i0 == e, v0, 0.0) + jnp.where(i1 == e, v1, 0.0)
                xm = (xs * m).astype(jnp.bfloat16)
                c = jnp.dot(xm, ew_bf[k], preferred_element_type=jnp.float32)
                acc = c if acc is None else acc + c
            pbuf[ro : ro + nrows, :] = acc.astype(jnp.bfloat16)

        def sem_idx(t, s, c):
            return N_CSUB * (N_STREAMS * t + s) + c

        def xc_rdma(t, s, c):
            ro, nrows = STREAMS[s]
            cs = pl.ds(c * CSUB, CSUB)
            return pltpu.make_async_remote_copy(
                src_ref=pbuf.at[pl.ds(ro, nrows), cs],
                dst_ref=rbuf.at[pl.ds(RBUF_BASE[s][t], nrows), cs],
                send_sem=send_sems.at[sem_idx(t, s, c)],
                recv_sem=recv_sems.at[sem_idx(t, s, c)],
                device_id=(partners[orders[s][t]],),
                device_id_type=pltpu.DeviceIdType.MESH,
            )

        def xc_add(t, s, c):
            ro, nrows = STREAMS[s]
            cs = pl.ds(c * CSUB, CSUB)
            total = (
                pbuf[pl.ds(ro, nrows), cs]
                + rbuf[pl.ds(RBUF_BASE[s][t], nrows), cs]
            )
            if t < 2:
                pbuf[pl.ds(ro, nrows), cs] = total
            else:
                out_ref[pl.ds(ro, nrows), cs] = total.astype(jnp.float32)

        compute_stream(0)
        if _KMODE == "withbarrier":
            pltpu.semaphore_wait(barrier_sem, 3)

        if _KMODE in ("nocomm", "nobarrier"):
            compute_stream(1)
            compute_stream(2)
            out_ref[:, :] = pbuf[:, :].astype(jnp.float32)
            return

        cur = {}
        for s in range(N_STREAMS):
            if s > 0:
                compute_stream(s)
            for c in range(N_CSUB):
                cur[(s, c)] = xc_rdma(0, s, c)
                cur[(s, c)].start()

        for t in range(3):
            nxt = {}
            for sc in SUBS:
                cur[sc].wait()
                xc_add(t, *sc)
                if t < 2:
                    nxt[sc] = xc_rdma(t + 1, *sc)
                    nxt[sc].start()
            cur = nxt

    return pl.pallas_call(
        body,
        out_shape=jax.ShapeDtypeStruct((ROWS, D_FF), jnp.float32),
        in_specs=[pl.BlockSpec(memory_space=pltpu.VMEM)] * 4,
        out_specs=pl.BlockSpec(memory_space=pltpu.VMEM),
        scratch_shapes=[
            pltpu.VMEM((ROWS, D_FF), jnp.bfloat16),
            pltpu.VMEM((RBUF_ROWS, D_FF), jnp.bfloat16),
            pltpu.SemaphoreType.DMA((36,)),
            pltpu.SemaphoreType.DMA((36,)),
        ],
        compiler_params=(
            pltpu.CompilerParams()
            if _KMODE in ("min", "nobarrier")
            else pltpu.CompilerParams(collective_id=0)
        ),
    )(x, router_W, route_idx, expert_W)


# device time: 17881 ns/iter; 1.0632x vs baseline; 1.0632x over previous
import os

import jax
import jax.numpy as jnp
from jax import lax
from jax.experimental import pallas as pl
from jax.experimental.pallas import tpu as pltpu

_KMODE = os.environ.get("KMODE", "full")

N_DEV = 8
ROWS = 512
D_MODEL = 256
D_FF = 512
N_EXP = 16
EXP_PER_DEV = N_EXP // N_DEV
N_CSUB = 4
CSUB = D_FF // N_CSUB

STREAMS = ((0, 160), (160, 176), (336, 176))
N_STREAMS = 3
RBUF_BASE = []
_off = 0
for _b, _r in STREAMS:
    RBUF_BASE.append(tuple(_off + t * _r for t in range(3)))
    _off += 3 * _r
RBUF_ROWS = _off

SUBS = tuple((s, c) for c in range(N_CSUB) for s in range(N_STREAMS))


def kernel(x, router_W, route_idx, expert_W):
    def body(
        x_ref,
        rw_ref,
        idx_ref,
        ew_ref,
        out_ref,
        pbuf,
        rbuf,
        send_sems,
        recv_sems,
    ):
        my = lax.axis_index("i")
        partners = (my ^ 1, my ^ 3, my ^ 4)
        orders = ((0, 1, 2), (1, 2, 0), (2, 0, 1))

        if _KMODE == "min":
            out_ref[:, :] = jnp.zeros((ROWS, D_FF), jnp.float32)
            return

        if _KMODE != "nobarrier":
            barrier_sem = pltpu.get_barrier_semaphore()
            if _KMODE == "nosync":
                pltpu.semaphore_signal(barrier_sem, inc=1)
                pltpu.semaphore_wait(barrier_sem, 1)
            else:
                for prt in partners:
                    pltpu.semaphore_signal(
                        barrier_sem,
                        inc=1,
                        device_id=(prt,),
                        device_id_type=pltpu.DeviceIdType.MESH,
                    )

        x_v = x_ref[:, :]
        scores = jnp.dot(x_v, rw_ref[:, :], preferred_element_type=jnp.float32)
        s_max = jnp.max(scores, axis=-1, keepdims=True)
        p = jnp.exp(scores - s_max)
        p = p / jnp.sum(p, axis=-1, keepdims=True)

        idx0 = idx_ref[:, 0:1]
        idx1 = idx_ref[:, 1:2]
        e_iota = lax.broadcasted_iota(jnp.int32, (ROWS, N_EXP), 1)
        g0 = jnp.sum(jnp.where(idx0 == e_iota, p, 0.0), axis=-1, keepdims=True)
        g1 = jnp.sum(jnp.where(idx1 == e_iota, p, 0.0), axis=-1, keepdims=True)
        gs = g0 + g1
        w0 = g0 / gs
        w1 = g1 / gs

        e_base = my * EXP_PER_DEV
        ew_bf = [
            ew_ref[k, :, :].astype(jnp.bfloat16) for k in range(EXP_PER_DEV)
        ]

        def compute_stream(s):
            ro, nrows = STREAMS[s]
            xs = x_v[ro : ro + nrows, :]
            i0 = idx0[ro : ro + nrows, :]
            i1 = idx1[ro : ro + nrows, :]
            v0 = w0[ro : ro + nrows, :]
            v1 = w1[ro : ro + nrows, :]
            acc = None
            for k in range(EXP_PER_DEV):
                e = e_base + k
                m = jnp.where(i0 == e, v0, 0.0) + jnp.where(i1 == e, v1, 0.0)
                xm = (xs * m).astype(jnp.bfloat16)
                c = jnp.dot(xm, ew_bf[k], preferred_element_type=jnp.float32)
                acc = c if acc is None else acc + c
            pbuf[ro : ro + nrows, :] = acc.astype(jnp.bfloat16)

        def sem_idx(t, s, c):
            return N_CSUB * (N_STREAMS * t + s) + c

        def xc_rdma(t, s, c):
            ro, nrows = STREAMS[s]
            cs = pl.ds(c * CSUB, CSUB)
            return pltpu.make_async_remote_copy(
                src_ref=pbuf.at[pl.ds(ro, nrows), cs],
                dst_ref=rbuf.at[pl.ds(RBUF_BASE[s][t], nrows), cs],
                send_sem=send_sems.at[sem_idx(t, s, c)],
                recv_sem=recv_sems.at[sem_idx(t, s, c)],
                device_id=(partners[orders[s][t]],),
                device_id_type=pltpu.DeviceIdType.MESH,
            )

        def xc_add(t, s, c):
            ro, nrows = STREAMS[s]
            cs = pl.ds(c * CSUB, CSUB)
            total = (
                pbuf[pl.ds(ro, nrows), cs]
                + rbuf[pl.ds(RBUF_BASE[s][t], nrows), cs]
            )
            if t < 2:
                pbuf[pl.ds(ro, nrows), cs] = total
            else:
                out_ref[pl.ds(ro, nrows), cs] = total.astype(jnp.float32)

        compute_stream(0)
        if _KMODE not in ("nobarrier", "nosync"):
            pltpu.semaphore_wait(barrier_sem, 3)

        if _KMODE in ("nocomm", "nobarrier"):
            compute_stream(1)
            compute_stream(2)
            out_ref[:, :] = pbuf[:, :].astype(jnp.float32)
            return

        cur = {}
        for s in range(N_STREAMS):
            if s > 0:
                compute_stream(s)
            for c in range(N_CSUB):
                cur[(s, c)] = xc_rdma(0, s, c)
                cur[(s, c)].start()

        for t in range(3):
            nxt = {}
            for sc in SUBS:
                cur[sc].wait()
                xc_add(t, *sc)
                if t < 2:
                    nxt[sc] = xc_rdma(t + 1, *sc)
                    nxt[sc].start()
            cur = nxt

    return pl.pallas_call(
        body,
        out_shape=jax.ShapeDtypeStruct((ROWS, D_FF), jnp.float32),
        in_specs=[pl.BlockSpec(memory_space=pltpu.VMEM)] * 4,
        out_specs=pl.BlockSpec(memory_space=pltpu.VMEM),
        scratch_shapes=[
            pltpu.VMEM((ROWS, D_FF), jnp.bfloat16),
            pltpu.VMEM((RBUF_ROWS, D_FF), jnp.bfloat16),
            pltpu.SemaphoreType.DMA((36,)),
            pltpu.SemaphoreType.DMA((36,)),
        ],
        compiler_params=(
            pltpu.CompilerParams()
            if _KMODE in ("min", "nobarrier")
            else pltpu.CompilerParams(collective_id=0)
        ),
    )(x, router_W, route_idx, expert_W)


# device time: 17879 ns/iter; 1.0633x vs baseline; 1.0001x over previous
import os

import jax
import jax.numpy as jnp
from jax import lax
from jax.experimental import pallas as pl
from jax.experimental.pallas import tpu as pltpu

_KMODE = os.environ.get("KMODE", "full")

N_DEV = 8
ROWS = 512
D_MODEL = 256
D_FF = 512
N_EXP = 16
EXP_PER_DEV = N_EXP // N_DEV
N_CSUB = 4
CSUB = D_FF // N_CSUB

STREAMS = ((0, 160), (160, 176), (336, 176))
N_STREAMS = 3
RBUF_BASE = []
_off = 0
for _b, _r in STREAMS:
    RBUF_BASE.append(tuple(_off + t * _r for t in range(3)))
    _off += 3 * _r
RBUF_ROWS = _off

SUBS = tuple((s, c) for c in range(N_CSUB) for s in range(N_STREAMS))


def kernel(x, router_W, route_idx, expert_W):
    def body(
        x_ref,
        rw_ref,
        idx_ref,
        ew_ref,
        out_ref,
        pbuf,
        rbuf,
        send_sems,
        recv_sems,
    ):
        my = lax.axis_index("i")
        partners = (my ^ 1, my ^ 3, my ^ 4)
        orders = ((0, 1, 2), (1, 2, 0), (2, 0, 1))

        if _KMODE in ("min", "minany"):
            out_ref[:, :] = jnp.zeros((ROWS, D_FF), jnp.float32)
            return

        if _KMODE != "nobarrier":
            barrier_sem = pltpu.get_barrier_semaphore()
            if _KMODE == "nosync":
                pltpu.semaphore_signal(barrier_sem, inc=1)
                pltpu.semaphore_wait(barrier_sem, 1)
            else:
                for prt in partners:
                    pltpu.semaphore_signal(
                        barrier_sem,
                        inc=1,
                        device_id=(prt,),
                        device_id_type=pltpu.DeviceIdType.MESH,
                    )

        x_v = x_ref[:, :]
        scores = jnp.dot(x_v, rw_ref[:, :], preferred_element_type=jnp.float32)
        s_max = jnp.max(scores, axis=-1, keepdims=True)
        p = jnp.exp(scores - s_max)
        p = p / jnp.sum(p, axis=-1, keepdims=True)

        idx0 = idx_ref[:, 0:1]
        idx1 = idx_ref[:, 1:2]
        e_iota = lax.broadcasted_iota(jnp.int32, (ROWS, N_EXP), 1)
        g0 = jnp.sum(jnp.where(idx0 == e_iota, p, 0.0), axis=-1, keepdims=True)
        g1 = jnp.sum(jnp.where(idx1 == e_iota, p, 0.0), axis=-1, keepdims=True)
        gs = g0 + g1
        w0 = g0 / gs
        w1 = g1 / gs

        e_base = my * EXP_PER_DEV
        ew_bf = [
            ew_ref[k, :, :].astype(jnp.bfloat16) for k in range(EXP_PER_DEV)
        ]

        def compute_stream(s):
            ro, nrows = STREAMS[s]
            xs = x_v[ro : ro + nrows, :]
            i0 = idx0[ro : ro + nrows, :]
            i1 = idx1[ro : ro + nrows, :]
            v0 = w0[ro : ro + nrows, :]
            v1 = w1[ro : ro + nrows, :]
            acc = None
            for k in range(EXP_PER_DEV):
                e = e_base + k
                m = jnp.where(i0 == e, v0, 0.0) + jnp.where(i1 == e, v1, 0.0)
                xm = (xs * m).astype(jnp.bfloat16)
                c = jnp.dot(xm, ew_bf[k], preferred_element_type=jnp.float32)
                acc = c if acc is None else acc + c
            pbuf[ro : ro + nrows, :] = acc.astype(jnp.bfloat16)

        def sem_idx(t, s, c):
            return N_CSUB * (N_STREAMS * t + s) + c

        def xc_rdma(t, s, c):
            ro, nrows = STREAMS[s]
            cs = pl.ds(c * CSUB, CSUB)
            return pltpu.make_async_remote_copy(
                src_ref=pbuf.at[pl.ds(ro, nrows), cs],
                dst_ref=rbuf.at[pl.ds(RBUF_BASE[s][t], nrows), cs],
                send_sem=send_sems.at[sem_idx(t, s, c)],
                recv_sem=recv_sems.at[sem_idx(t, s, c)],
                device_id=(partners[orders[s][t]],),
                device_id_type=pltpu.DeviceIdType.MESH,
            )

        def xc_add(t, s, c):
            ro, nrows = STREAMS[s]
            cs = pl.ds(c * CSUB, CSUB)
            total = (
                pbuf[pl.ds(ro, nrows), cs]
                + rbuf[pl.ds(RBUF_BASE[s][t], nrows), cs]
            )
            if t < 2:
                pbuf[pl.ds(ro, nrows), cs] = total
            else:
                out_ref[pl.ds(ro, nrows), cs] = total.astype(jnp.float32)

        compute_stream(0)
        if _KMODE not in ("nobarrier", "nosync"):
            pltpu.semaphore_wait(barrier_sem, 3)

        if _KMODE in ("nocomm", "nobarrier"):
            compute_stream(1)
            compute_stream(2)
            out_ref[:, :] = pbuf[:, :].astype(jnp.float32)
            return

        cur = {}
        for s in range(N_STREAMS):
            if s > 0:
                compute_stream(s)
            for c in range(N_CSUB):
                cur[(s, c)] = xc_rdma(0, s, c)
                cur[(s, c)].start()

        for t in range(3):
            nxt = {}
            for sc in SUBS:
                cur[sc].wait()
                xc_add(t, *sc)
                if t < 2:
                    nxt[sc] = xc_rdma(t + 1, *sc)
                    nxt[sc].start()
            cur = nxt

    return pl.pallas_call(
        body,
        out_shape=jax.ShapeDtypeStruct((ROWS, D_FF), jnp.float32),
        in_specs=(
            [pl.BlockSpec(memory_space=pl.ANY)] * 4
            if _KMODE == "minany"
            else [pl.BlockSpec(memory_space=pltpu.VMEM)] * 4
        ),
        out_specs=pl.BlockSpec(memory_space=pltpu.VMEM),
        scratch_shapes=[
            pltpu.VMEM((ROWS, D_FF), jnp.bfloat16),
            pltpu.VMEM((RBUF_ROWS, D_FF), jnp.bfloat16),
            pltpu.SemaphoreType.DMA((36,)),
            pltpu.SemaphoreType.DMA((36,)),
        ],
        compiler_params=(
            pltpu.CompilerParams()
            if _KMODE in ("min", "minany", "nobarrier")
            else pltpu.CompilerParams(collective_id=0)
        ),
    )(x, router_W, route_idx, expert_W)
